# baseline (speedup 1.0000x reference)
import functools

import jax
import jax.numpy as jnp
from jax import lax
from jax.experimental import pallas as pl
from jax.experimental.pallas import tpu as pltpu
from jax.experimental.pallas import tpu_sc as plsc

_BF = jnp.bfloat16
_F32 = jnp.float32


def _conv_body(ph_ref, w_ref, out_ref, *, taps, n):
    acc = None
    for t, (p, off) in enumerate(taps):
        b = ph_ref[0, p, :, off:off + n]
        contrib = jnp.dot(w_ref[t], b, preferred_element_type=_F32)
        acc = contrib if acc is None else acc + contrib
    out_ref[0] = jnp.maximum(acc, 0.0)


def _conv_call(phases, w_taps, taps, n):
    bsz, pnum, ci, lp = phases.shape
    tnum, co, _ = w_taps.shape
    return pl.pallas_call(
        functools.partial(_conv_body, taps=taps, n=n),
        grid=(bsz,),
        in_specs=[
            pl.BlockSpec((1, pnum, ci, lp), lambda b: (b, 0, 0, 0)),
            pl.BlockSpec((tnum, co, ci), lambda b: (0, 0, 0)),
        ],
        out_specs=pl.BlockSpec((1, co, n), lambda b: (b, 0, 0)),
        out_shape=jax.ShapeDtypeStruct((bsz, co, n), _F32),
    )(phases, w_taps)


def _conv0_body(ph_ref, w_ref, out_ref, *, n):
    parts = []
    for ky in range(7):
        for kx in range(7):
            p = (ky % 4) * 4 + (kx % 4)
            off = (ky // 4) * 98 + (kx // 4)
            parts.append(ph_ref[0, p, :, off:off + n])
    b = jnp.concatenate(parts, axis=0)
    out_ref[0] = jnp.maximum(jnp.dot(w_ref[...], b, preferred_element_type=_F32), 0.0)


def _conv0_call(phases, w_mat, n):
    bsz, pnum, ci, lp = phases.shape
    co, _ = w_mat.shape
    return pl.pallas_call(
        functools.partial(_conv0_body, n=n),
        grid=(bsz,),
        in_specs=[
            pl.BlockSpec((1, pnum, ci, lp), lambda b: (b, 0, 0, 0)),
            pl.BlockSpec(w_mat.shape, lambda b: (0, 0)),
        ],
        out_specs=pl.BlockSpec((1, co, n), lambda b: (b, 0, 0)),
        out_shape=jax.ShapeDtypeStruct((bsz, co, n), _F32),
    )(phases, w_mat)


def _phase_split(xp, s):
    bsz, c, hp, wp = xp.shape
    ph = jnp.stack(
        [xp[:, :, pr::s, pc::s] for pr in range(s) for pc in range(s)], axis=1
    )
    return ph.reshape(bsz, s * s, c, (hp // s) * (wp // s))


def _lane_pad(a, lp):
    return jnp.pad(a, [(0, 0)] * (a.ndim - 1) + [(0, lp - a.shape[-1])])


_NC, _NS = 2, 16
_NW = _NC * _NS


def _gather_rows(table, idx, chunk):
    rows, d = table.shape
    bsz = idx.shape[0]
    b_per_w = bsz // _NW
    nchunks = b_per_w // chunk
    idx3 = idx.reshape(_NW, nchunks, chunk)
    mesh = plsc.VectorSubcoreMesh(core_axis_name="c", subcore_axis_name="s")

    @functools.partial(
        pl.kernel,
        mesh=mesh,
        out_type=jax.ShapeDtypeStruct((bsz, d), _F32),
        scratch_types=[
            pltpu.VMEM((chunk,), jnp.int32),
            pltpu.VMEM((chunk, d), _F32),
            pltpu.SemaphoreType.DMA,
        ],
    )
    def k(table_hbm, idx_hbm, out_hbm, idx_v, rows_v, sem):
        cid = lax.axis_index("c")
        sid = lax.axis_index("s")
        wid = sid * _NC + cid
        for c in range(nchunks):
            pltpu.sync_copy(idx_hbm.at[wid, c], idx_v)
            pltpu.async_copy(table_hbm.at[idx_v], rows_v, sem).wait()
            pltpu.sync_copy(
                rows_v, out_hbm.at[pl.ds(wid * b_per_w + c * chunk, chunk)]
            )

    return k(table, idx3)


def kernel(x, W0, W1, W2, W3, W4, fm0, fm1):
    bsz = x.shape[0]

    xp = jnp.pad(x, ((0, 0), (0, 0), (3, 5), (3, 5)))
    ph0 = _phase_split(xp, 4).astype(_BF)
    a0 = W0.transpose(0, 2, 3, 1).reshape(96, 147).astype(_BF)
    h_slab = _conv0_call(ph0, a0, 96 * 98)
    h = h_slab.reshape(bsz, 96, 96, 98)[..., :96]

    hp = jnp.pad(h, ((0, 0), (0, 0), (1, 1), (1, 1)))
    ph1 = _lane_pad(_phase_split(hp, 2), 2408).astype(_BF)
    w1 = W1.transpose(2, 3, 0, 1).astype(_BF)
    taps1 = [((ky % 2) * 2 + (kx % 2), (ky // 2) * 49 + (kx // 2))
             for ky in range(3) for kx in range(3)]
    f0_slab = _conv_call(ph1, w1.reshape(9, 192, 96), taps1, 48 * 49)
    f0 = f0_slab.reshape(bsz, 192, 48, 49)[..., :48]

    f0p = jnp.pad(f0, ((0, 0), (0, 0), (1, 1), (1, 1)))
    in2 = _lane_pad(f0p.reshape(bsz, 1, 192, 2500), 2504).astype(_BF)
    w2 = W2.transpose(2, 3, 0, 1).astype(_BF)
    taps2 = [(0, ky * 50 + kx) for ky in range(3) for kx in range(3)]
    f1_slab = _conv_call(in2, w2.reshape(9, 192, 192), taps2, 48 * 50)
    f1 = jnp.pad(f1_slab, ((0, 0), (0, 0), (51, 49))).reshape(
        bsz, 192, 50, 50)[:, :, 1:49, 1:49]

    f1p = jnp.pad(f1, ((0, 0), (0, 0), (1, 1), (1, 1)))
    ph3 = _lane_pad(_phase_split(f1p, 2), 632).astype(_BF)
    w3 = W3.transpose(2, 3, 0, 1).astype(_BF)
    taps3 = [((ky % 2) * 2 + (kx % 2), (ky // 2) * 25 + (kx // 2))
             for ky in range(3) for kx in range(3)]
    f2_slab = _conv_call(ph3, w3.reshape(9, 384, 192), taps3, 24 * 25)
    f2 = f2_slab.reshape(bsz, 384, 24, 25)[..., :24]

    f2p = jnp.pad(f2, ((0, 0), (0, 0), (1, 1), (1, 1)))
    in4 = _lane_pad(f2p.reshape(bsz, 1, 384, 676), 680).astype(_BF)
    w4 = W4.transpose(2, 3, 0, 1).astype(_BF)
    taps4 = [(0, ky * 26 + kx) for ky in range(3) for kx in range(3)]
    f3_slab = _conv_call(in4, w4.reshape(9, 384, 384), taps4, 24 * 26)
    f3 = jnp.pad(f3_slab, ((0, 0), (0, 0), (27, 25))).reshape(
        bsz, 384, 26, 26)[:, :, 1:25, 1:25]

    cat0 = jnp.concatenate([f0, f1], axis=1).reshape(bsz * 384, 48 * 48)
    cat1 = jnp.pad(
        jnp.concatenate([f2, f3], axis=1).reshape(bsz * 768, 24 * 24),
        ((0, 0), (0, 64)))
    idx0 = (jnp.arange(bsz, dtype=jnp.int32)[:, None] * 384 + fm0[None, :]
            ).reshape(-1)
    idx1 = (jnp.arange(bsz, dtype=jnp.int32)[:, None] * 768 + fm1[None, :]
            ).reshape(-1)
    out0 = jnp.take(cat0, idx0, axis=0).reshape(bsz, fm0.shape[0], 48, 48)
    out1 = jnp.take(cat1, idx1, axis=0)[:, :576].reshape(
        bsz, fm1.shape[0], 24, 24)
    return (out0, out1)


# speedup vs baseline: 1.0715x; 1.0715x over previous
import functools

import jax
import jax.numpy as jnp
from jax import lax
from jax.experimental import pallas as pl
from jax.experimental.pallas import tpu as pltpu
from jax.experimental.pallas import tpu_sc as plsc

_BF = jnp.bfloat16
_F32 = jnp.float32


def _k0_body(ph_ref, w_ref, mask_ref, out_ref):
    parts = []
    for ky in range(7):
        for kx in range(7):
            p = (ky % 4) * 4 + (kx % 4)
            off = (ky // 4) * 104 + (kx // 4)
            parts.append(ph_ref[0, p, :, off:off + 9984])
    b = jnp.concatenate(parts, axis=0)
    val = jnp.dot(w_ref[...], b, preferred_element_type=_F32)
    val = jnp.maximum(val, 0.0) * mask_ref[...]
    t = jnp.transpose(val.astype(_BF))
    out_ref[0, 0:105, :] = jnp.zeros((105, 128), _BF)
    out_ref[0, 105:10089, :] = t
    out_ref[0, 10089:10192, :] = jnp.zeros((103, 128), _BF)


def _k0_call(phases, w_mat, mask):
    bsz = phases.shape[0]
    return pl.pallas_call(
        _k0_body,
        grid=(bsz,),
        in_specs=[
            pl.BlockSpec((1, 16, 3, 10192), lambda b: (b, 0, 0, 0)),
            pl.BlockSpec((128, 147), lambda b: (0, 0)),
            pl.BlockSpec((1, 9984), lambda b: (0, 0)),
        ],
        out_specs=pl.BlockSpec((1, 10192, 128), lambda b: (b, 0, 0)),
        out_shape=jax.ShapeDtypeStruct((bsz, 10192, 128), _BF),
    )(phases, w_mat, mask)


def _conv_body(in_ref, w_ref, mask_ref, can_ref, cl_ref, scr_ref, *, geom,
               co_real):
    (hin, win, wpad, stride, hq, wq, wcont, hcan, wcan) = geom
    ci = in_ref.shape[3]
    g = ci // 128
    co = w_ref.shape[2]
    scr_ref[:, 0:win, :, :] = in_ref[0].astype(_F32).reshape(
        hin, win, g, 128)
    scr_ref[:, win:wpad, :, :] = jnp.zeros((hin, wpad - win, g, 128), _F32)
    acc = None
    for ky in range(3):
        for kx in range(3):
            a = scr_ref[ky:ky + stride * (hq - 1) + 1:stride,
                        kx:kx + stride * (wq - 1) + 1:stride, :, :]
            a = a.astype(_BF).reshape(hq * wq, ci)
            d = jnp.dot(a, w_ref[ky * 3 + kx], preferred_element_type=_F32)
            acc = d if acc is None else acc + d
    r = jnp.maximum(acc, 0.0)
    if cl_ref is not None:
        clean = jnp.concatenate(
            [r[i * wq:i * wq + wcont, 0:co_real] for i in range(hq)], axis=0)
        cl_ref[0] = clean
    rm = (r * mask_ref[...]).astype(_BF)
    can_ref[0, 0:wcan + 1, :] = jnp.zeros((wcan + 1, co), _BF)
    can_ref[0, wcan + 1:wcan + 1 + hq * wq, :] = rm
    tail = hcan * wcan - (wcan + 1 + hq * wq)
    can_ref[0, wcan + 1 + hq * wq:, :] = jnp.zeros((tail, co), _BF)


def _conv_call(inp, w_taps, mask, geom, co_real):
    bsz = inp.shape[0]
    (hin, win, wpad, stride, hq, wq, wcont, hcan, wcan) = geom
    ci = inp.shape[2]
    co = w_taps.shape[2]
    with_clean = co_real is not None
    outs = [jax.ShapeDtypeStruct((bsz, hcan * wcan, co), _BF)]
    out_specs = [pl.BlockSpec((1, hcan * wcan, co), lambda b: (b, 0, 0))]
    if with_clean:
        outs.append(jax.ShapeDtypeStruct((bsz, hq * wcont, co_real), _F32))
        out_specs.append(
            pl.BlockSpec((1, hq * wcont, co_real), lambda b: (b, 0, 0)))

    def body(in_ref, w_ref, mask_ref, can_ref, *rest):
        if with_clean:
            cl_ref, scr_ref = rest
        else:
            cl_ref, scr_ref = None, rest[0]
        _conv_body(in_ref, w_ref, mask_ref, can_ref, cl_ref, scr_ref,
                   geom=geom, co_real=co_real)

    return pl.pallas_call(
        body,
        grid=(bsz,),
        in_specs=[
            pl.BlockSpec((1, hin, win, ci), lambda b: (b, 0, 0, 0)),
            pl.BlockSpec((9, ci, co), lambda b: (0, 0, 0)),
            pl.BlockSpec((hq * wq, 1), lambda b: (0, 0)),
        ],
        out_specs=out_specs,
        out_shape=outs,
        scratch_shapes=[pltpu.VMEM((hin, wpad, ci // 128, 128), _F32)],
    )(inp.reshape(bsz, hin, win, ci), w_taps, mask)


def _k4_body(in_ref, w_ref, out_ref):
    v = in_ref[0].reshape(26, 32, 384)
    acc = None
    for ky in range(3):
        for kx in range(3):
            a = v[ky:ky + 24, kx:kx + 24, :].reshape(576, 384)
            d = jnp.dot(a, w_ref[ky * 3 + kx], preferred_element_type=_F32)
            acc = d if acc is None else acc + d
    out_ref[0] = jnp.maximum(acc, 0.0)


def _k4_call(inp, w_taps):
    bsz = inp.shape[0]
    return pl.pallas_call(
        _k4_body,
        grid=(bsz,),
        in_specs=[
            pl.BlockSpec((1, 26 * 32, 384), lambda b: (b, 0, 0)),
            pl.BlockSpec((9, 384, 384), lambda b: (0, 0, 0)),
        ],
        out_specs=pl.BlockSpec((1, 576, 384), lambda b: (b, 0, 0)),
        out_shape=jax.ShapeDtypeStruct((bsz, 576, 384), _F32),
    )(inp, w_taps)


def _t_body(a_ref, b_ref, out_ref, *, dpad):
    npix = a_ref.shape[1]
    co = a_ref.shape[2]
    ta = jnp.transpose(a_ref[0])
    tb = jnp.transpose(b_ref[0])
    if dpad > npix:
        z = jnp.zeros((co, dpad - npix), _F32)
        ta = jnp.concatenate([ta, z], axis=1)
        tb = jnp.concatenate([tb, z], axis=1)
    out_ref[0, 0:co, :] = ta
    out_ref[0, co:2 * co, :] = tb


def _t_call(a, b, dpad):
    bsz, npix, co = a.shape
    return pl.pallas_call(
        functools.partial(_t_body, dpad=dpad),
        grid=(bsz,),
        in_specs=[
            pl.BlockSpec((1, npix, co), lambda i: (i, 0, 0)),
            pl.BlockSpec((1, npix, co), lambda i: (i, 0, 0)),
        ],
        out_specs=pl.BlockSpec((1, 2 * co, dpad), lambda i: (i, 0, 0)),
        out_shape=jax.ShapeDtypeStruct((bsz, 2 * co, dpad), _F32),
    )(a, b)


_NC, _NS = 2, 16
_NW = _NC * _NS


def _gather_rows(table, idx, chunk):
    rows, d = table.shape
    bsz = idx.shape[0]
    b_per_w = bsz // _NW
    nchunks = b_per_w // chunk
    idx3 = idx.reshape(_NW, nchunks, chunk)
    mesh = plsc.VectorSubcoreMesh(core_axis_name="c", subcore_axis_name="s")

    @functools.partial(
        pl.kernel,
        mesh=mesh,
        out_type=jax.ShapeDtypeStruct((bsz, d), _F32),
        scratch_types=[
            pltpu.VMEM((chunk,), jnp.int32),
            pltpu.VMEM((chunk, d), _F32),
            pltpu.SemaphoreType.DMA,
        ],
    )
    def k(table_hbm, idx_hbm, out_hbm, idx_v, rows_v, sem):
        cid = lax.axis_index("c")
        sid = lax.axis_index("s")
        wid = sid * _NC + cid
        for c in range(nchunks):
            pltpu.sync_copy(idx_hbm.at[wid, c], idx_v)
            pltpu.async_copy(table_hbm.at[idx_v], rows_v, sem).wait()
            pltpu.sync_copy(
                rows_v, out_hbm.at[pl.ds(wid * b_per_w + c * chunk, chunk)]
            )

    return k(table, idx3)


def _row_mask(hq, wq, wcont):
    m = jnp.zeros((hq, wq, 1), _F32).at[:, :wcont, :].set(1.0)
    return m.reshape(hq * wq, 1)


def kernel(x, W0, W1, W2, W3, W4, fm0, fm1):
    bsz = x.shape[0]

    xp = jnp.pad(x, ((0, 0), (0, 0), (3, 5), (3, 29)))
    ph0 = jnp.stack(
        [xp[:, :, pr::4, pc::4] for pr in range(4) for pc in range(4)],
        axis=1).reshape(bsz, 16, 3, 10192).astype(_BF)
    a0 = jnp.pad(W0.transpose(0, 2, 3, 1).reshape(96, 147),
                 ((0, 32), (0, 0))).astype(_BF)
    mask0 = (jnp.arange(9984, dtype=jnp.int32)[None, :] % 104 < 96
             ).astype(_F32)
    hc = _k0_call(ph0, a0, mask0)

    w1 = jnp.pad(W1.transpose(2, 3, 1, 0).reshape(9, 96, 192),
                 ((0, 0), (0, 32), (0, 64))).astype(_BF)
    w2 = jnp.pad(W2.transpose(2, 3, 1, 0).reshape(9, 192, 192),
                 ((0, 0), (0, 64), (0, 64))).astype(_BF)
    w3 = jnp.pad(W3.transpose(2, 3, 1, 0).reshape(9, 192, 384),
                 ((0, 0), (0, 64), (0, 0))).astype(_BF)
    w4 = W4.transpose(2, 3, 1, 0).reshape(9, 384, 384).astype(_BF)

    g1 = (98, 104, 120, 2, 48, 56, 48, 50, 56)
    f0can, f0cl = _conv_call(hc, w1, _row_mask(48, 56, 48), g1, 192)

    g2 = (50, 56, 64, 1, 48, 56, 48, 50, 56)
    f1can, f1cl = _conv_call(f0can, w2, _row_mask(48, 56, 48), g2, 192)

    g3 = (50, 56, 72, 2, 24, 32, 24, 26, 32)
    f2can, f2cl = _conv_call(f1can, w3, _row_mask(24, 32, 24), g3, 384)

    f3cl = _k4_call(f2can, w4)

    cat0 = _t_call(f0cl, f1cl, 2304).reshape(bsz * 384, 2304)
    cat1 = _t_call(f2cl, f3cl, 640).reshape(bsz * 768, 640)
    idx0 = (jnp.arange(bsz, dtype=jnp.int32)[:, None] * 384 + fm0[None, :]
            ).reshape(-1)
    idx1 = (jnp.arange(bsz, dtype=jnp.int32)[:, None] * 768 + fm1[None, :]
            ).reshape(-1)
    out0 = _gather_rows(cat0, idx0, 16).reshape(bsz, fm0.shape[0], 48, 48)
    out1 = _gather_rows(cat1, idx1, 64)[:, :576].reshape(
        bsz, fm1.shape[0], 24, 24)
    return (out0, out1)


# speedup vs baseline: 2.6863x; 2.5071x over previous
import functools

import jax
import jax.numpy as jnp
from jax import lax
from jax.experimental import pallas as pl
from jax.experimental.pallas import tpu as pltpu
from jax.experimental.pallas import tpu_sc as plsc

_BF = jnp.bfloat16
_F32 = jnp.float32


def _mega_body(ph_ref, a0_ref, w1_ref, w2_ref, w3_ref, w4_ref,
               cat0_ref, cat1_ref, sh, sf0, sf1, sf2):
    b = pl.program_id(0)

    @pl.when(b == 0)
    def _zero_canvases():
        sh[...] = jnp.zeros(sh.shape, _F32)
        sf0[...] = jnp.zeros(sf0.shape, _F32)
        sf1[...] = jnp.zeros(sf1.shape, _F32)
        sf2[...] = jnp.zeros(sf2.shape, _F32)

    parts = []
    for ky in range(7):
        for kx in range(7):
            p = (ky % 4) * 4 + (kx % 4)
            off = (ky // 4) * 104 + (kx // 4)
            parts.append(ph_ref[0, p, :, off:off + 9984])
    bb = jnp.concatenate(parts, axis=0)
    val = jnp.dot(a0_ref[...], bb, preferred_element_type=_F32)
    val = jnp.maximum(val, 0.0)
    t3 = jnp.transpose(val).reshape(96, 104, 128)
    sh[1:97, 1:97, :] = t3[:, 0:96, :]

    acc = None
    for ky in range(3):
        for kx in range(3):
            a = sh[ky:ky + 95:2, kx:kx + 95:2, :]
            a = a.astype(_BF).reshape(2304, 128)
            d = jnp.dot(a, w1_ref[ky * 3 + kx], preferred_element_type=_F32)
            acc = d if acc is None else acc + d
    r = jnp.maximum(acc, 0.0)
    cat0_ref[0, 0:192, :] = jnp.transpose(r[:, 0:192])
    sf0[1:49, 1:49, :, :] = r.reshape(48, 48, 2, 128)

    acc = None
    for ky in range(3):
        for kx in range(3):
            a = sf0[ky:ky + 48, kx:kx + 48, :, :]
            a = a.astype(_BF).reshape(2304, 256)
            d = jnp.dot(a, w2_ref[ky * 3 + kx], preferred_element_type=_F32)
            acc = d if acc is None else acc + d
    r = jnp.maximum(acc, 0.0)
    cat0_ref[0, 192:384, :] = jnp.transpose(r[:, 0:192])
    sf1[1:49, 1:49, :, :] = r.reshape(48, 48, 2, 128)

    acc = None
    for ky in range(3):
        for kx in range(3):
            a = sf1[ky:ky + 47:2, kx:kx + 47:2, :, :]
            a = a.astype(_BF).reshape(576, 256)
            d = jnp.dot(a, w3_ref[ky * 3 + kx], preferred_element_type=_F32)
            acc = d if acc is None else acc + d
    r = jnp.maximum(acc, 0.0)
    cat1_ref[0, :, 576:640] = jnp.zeros((768, 64), _F32)
    cat1_ref[0, 0:384, 0:576] = jnp.transpose(r)
    sf2[1:25, 1:25, :, :] = r.reshape(24, 24, 3, 128)

    acc = None
    for ky in range(3):
        for kx in range(3):
            a = sf2[ky:ky + 24, kx:kx + 24, :, :]
            a = a.astype(_BF).reshape(576, 384)
            d = jnp.dot(a, w4_ref[ky * 3 + kx], preferred_element_type=_F32)
            acc = d if acc is None else acc + d
    r = jnp.maximum(acc, 0.0)
    cat1_ref[0, 384:768, 0:576] = jnp.transpose(r)


def _mega_call(ph0, a0, w1, w2, w3, w4):
    bsz = ph0.shape[0]
    return pl.pallas_call(
        _mega_body,
        grid=(bsz,),
        in_specs=[
            pl.BlockSpec((1, 16, 3, 10192), lambda b: (b, 0, 0, 0)),
            pl.BlockSpec((128, 147), lambda b: (0, 0)),
            pl.BlockSpec((9, 128, 256), lambda b: (0, 0, 0)),
            pl.BlockSpec((9, 256, 256), lambda b: (0, 0, 0)),
            pl.BlockSpec((9, 256, 384), lambda b: (0, 0, 0)),
            pl.BlockSpec((9, 384, 384), lambda b: (0, 0, 0)),
        ],
        out_specs=[
            pl.BlockSpec((1, 384, 2304), lambda b: (b, 0, 0)),
            pl.BlockSpec((1, 768, 640), lambda b: (b, 0, 0)),
        ],
        out_shape=[
            jax.ShapeDtypeStruct((bsz, 384, 2304), _F32),
            jax.ShapeDtypeStruct((bsz, 768, 640), _F32),
        ],
        scratch_shapes=[
            pltpu.VMEM((98, 98, 128), _F32),
            pltpu.VMEM((50, 50, 2, 128), _F32),
            pltpu.VMEM((50, 50, 2, 128), _F32),
            pltpu.VMEM((26, 26, 3, 128), _F32),
        ],
    )(ph0, a0, w1, w2, w3, w4)


_NC, _NS = 2, 16
_NW = _NC * _NS


def _gather_rows(table, idx, chunk):
    rows, d = table.shape
    bsz = idx.shape[0]
    b_per_w = bsz // _NW
    nchunks = b_per_w // chunk
    idx3 = idx.reshape(_NW, nchunks, chunk)
    mesh = plsc.VectorSubcoreMesh(core_axis_name="c", subcore_axis_name="s")

    @functools.partial(
        pl.kernel,
        mesh=mesh,
        out_type=jax.ShapeDtypeStruct((bsz, d), _F32),
        scratch_types=[
            pltpu.VMEM((chunk,), jnp.int32),
            pltpu.VMEM((chunk, d), _F32),
            pltpu.SemaphoreType.DMA,
        ],
    )
    def k(table_hbm, idx_hbm, out_hbm, idx_v, rows_v, sem):
        cid = lax.axis_index("c")
        sid = lax.axis_index("s")
        wid = sid * _NC + cid
        for c in range(nchunks):
            pltpu.sync_copy(idx_hbm.at[wid, c], idx_v)
            pltpu.async_copy(table_hbm.at[idx_v], rows_v, sem).wait()
            pltpu.sync_copy(
                rows_v, out_hbm.at[pl.ds(wid * b_per_w + c * chunk, chunk)]
            )

    return k(table, idx3)


def kernel(x, W0, W1, W2, W3, W4, fm0, fm1):
    bsz = x.shape[0]

    xp = jnp.pad(x, ((0, 0), (0, 0), (3, 5), (3, 29)))
    ph0 = xp.reshape(bsz, 3, 98, 4, 104, 4).transpose(0, 3, 5, 1, 2, 4)
    ph0 = ph0.reshape(bsz, 16, 3, 10192).astype(_BF)

    a0 = jnp.pad(W0.transpose(0, 2, 3, 1).reshape(96, 147),
                 ((0, 32), (0, 0))).astype(_BF)
    w1 = jnp.pad(W1.transpose(2, 3, 1, 0).reshape(9, 96, 192),
                 ((0, 0), (0, 32), (0, 64))).astype(_BF)
    w2 = jnp.pad(W2.transpose(2, 3, 1, 0).reshape(9, 192, 192),
                 ((0, 0), (0, 64), (0, 64))).astype(_BF)
    w3 = jnp.pad(W3.transpose(2, 3, 1, 0).reshape(9, 192, 384),
                 ((0, 0), (0, 64), (0, 0))).astype(_BF)
    w4 = W4.transpose(2, 3, 1, 0).reshape(9, 384, 384).astype(_BF)

    cat0, cat1 = _mega_call(ph0, a0, w1, w2, w3, w4)
    cat0 = cat0.reshape(bsz * 384, 2304)
    cat1 = cat1.reshape(bsz * 768, 640)

    idx0 = (jnp.arange(bsz, dtype=jnp.int32)[:, None] * 384 + fm0[None, :]
            ).reshape(-1)
    idx1 = (jnp.arange(bsz, dtype=jnp.int32)[:, None] * 768 + fm1[None, :]
            ).reshape(-1)
    out0 = _gather_rows(cat0, idx0, 16).reshape(bsz, fm0.shape[0], 48, 48)
    out1 = _gather_rows(cat1, idx1, 64)[:, :576].reshape(
        bsz, fm1.shape[0], 24, 24)
    return (out0, out1)


# speedup vs baseline: 2.9308x; 1.0910x over previous
import functools

import jax
import jax.numpy as jnp
from jax import lax
from jax.experimental import pallas as pl
from jax.experimental.pallas import tpu as pltpu
from jax.experimental.pallas import tpu_sc as plsc

_BF = jnp.bfloat16
_F32 = jnp.float32


def _mega_body(ph_ref, a0_ref, w1_ref, w2_ref, w3_ref, w4_ref,
               cat0_ref, cat1_ref, sh, sf0, sf1, sf2):
    b = pl.program_id(0)

    @pl.when(b == 0)
    def _zero_canvases():
        sh[...] = jnp.zeros(sh.shape, _F32)
        sf0[...] = jnp.zeros(sf0.shape, _F32)
        sf1[...] = jnp.zeros(sf1.shape, _F32)
        sf2[...] = jnp.zeros(sf2.shape, _F32)

    parts = []
    for ky in range(7):
        for kx in range(7):
            p = (ky % 4) * 4 + (kx % 4)
            off = (ky // 4) * 104 + (kx // 4)
            parts.append(ph_ref[0, p, :, off:off + 9984])
    bb = jnp.concatenate(parts, axis=0)
    val = jnp.dot(a0_ref[...], bb, preferred_element_type=_F32)
    val = jnp.maximum(val, 0.0)
    t3 = jnp.transpose(val).reshape(96, 104, 128)
    sh[1:97, 1:97, :] = t3[:, 0:96, :]

    acc = None
    for ky in range(3):
        for kx in range(3):
            a = sh[ky:ky + 95:2, kx:kx + 95:2, :]
            a = a.astype(_BF).reshape(2304, 128)
            d = jnp.dot(a, w1_ref[ky * 3 + kx], preferred_element_type=_F32)
            acc = d if acc is None else acc + d
    r = jnp.maximum(acc, 0.0)
    cat0_ref[0, 0:192, :] = jnp.transpose(r[:, 0:192])
    sf0[1:49, 1:49, :, :] = r.reshape(48, 48, 2, 128)

    acc = None
    for ky in range(3):
        for kx in range(3):
            a = sf0[ky:ky + 48, kx:kx + 48, :, :]
            a = a.astype(_BF).reshape(2304, 256)
            d = jnp.dot(a, w2_ref[ky * 3 + kx], preferred_element_type=_F32)
            acc = d if acc is None else acc + d
    r = jnp.maximum(acc, 0.0)
    cat0_ref[0, 192:384, :] = jnp.transpose(r[:, 0:192])
    sf1[1:49, 1:49, :, :] = r.reshape(48, 48, 2, 128)

    acc = None
    for ky in range(3):
        for kx in range(3):
            a = sf1[ky:ky + 47:2, kx:kx + 47:2, :, :]
            a = a.astype(_BF).reshape(576, 256)
            d = jnp.dot(a, w3_ref[ky * 3 + kx], preferred_element_type=_F32)
            acc = d if acc is None else acc + d
    r = jnp.maximum(acc, 0.0)
    cat1_ref[0, :, 576:640] = jnp.zeros((768, 64), _F32)
    cat1_ref[0, 0:384, 0:576] = jnp.transpose(r)
    sf2[1:25, 1:25, :, :] = r.reshape(24, 24, 3, 128)

    acc = None
    for ky in range(3):
        for kx in range(3):
            a = sf2[ky:ky + 24, kx:kx + 24, :, :]
            a = a.astype(_BF).reshape(576, 384)
            d = jnp.dot(a, w4_ref[ky * 3 + kx], preferred_element_type=_F32)
            acc = d if acc is None else acc + d
    r = jnp.maximum(acc, 0.0)
    cat1_ref[0, 384:768, 0:576] = jnp.transpose(r)


def _mega_call(ph0, a0, w1, w2, w3, w4):
    bsz = ph0.shape[0]
    return pl.pallas_call(
        _mega_body,
        grid=(bsz,),
        in_specs=[
            pl.BlockSpec((1, 16, 3, 10192), lambda b: (b, 0, 0, 0)),
            pl.BlockSpec((128, 147), lambda b: (0, 0)),
            pl.BlockSpec((9, 128, 256), lambda b: (0, 0, 0)),
            pl.BlockSpec((9, 256, 256), lambda b: (0, 0, 0)),
            pl.BlockSpec((9, 256, 384), lambda b: (0, 0, 0)),
            pl.BlockSpec((9, 384, 384), lambda b: (0, 0, 0)),
        ],
        out_specs=[
            pl.BlockSpec((1, 384, 2304), lambda b: (b, 0, 0)),
            pl.BlockSpec((1, 768, 640), lambda b: (b, 0, 0)),
        ],
        out_shape=[
            jax.ShapeDtypeStruct((bsz, 384, 2304), _F32),
            jax.ShapeDtypeStruct((bsz, 768, 640), _F32),
        ],
        scratch_shapes=[
            pltpu.VMEM((98, 98, 128), _F32),
            pltpu.VMEM((50, 50, 2, 128), _F32),
            pltpu.VMEM((50, 50, 2, 128), _F32),
            pltpu.VMEM((26, 26, 3, 128), _F32),
        ],
    )(ph0, a0, w1, w2, w3, w4)


_NC, _NS = 2, 16
_NW = _NC * _NS


def _sc_gather_both(cat0, cat1, fm0, fm1, bsz):
    nf0, nf1 = fm0.shape[0], fm1.shape[0]
    b0, b1 = bsz * nf0, bsz * nf1
    pw0, pw1 = b0 // _NW, b1 // _NW
    c0, c1 = 16, 64
    mesh = plsc.VectorSubcoreMesh(core_axis_name="c", subcore_axis_name="s")

    @functools.partial(
        pl.kernel,
        mesh=mesh,
        out_type=[
            jax.ShapeDtypeStruct((b0, 2304), _F32),
            jax.ShapeDtypeStruct((b1, 640), _F32),
        ],
        scratch_types=[
            pltpu.VMEM((nf0,), jnp.int32),
            pltpu.VMEM((nf1,), jnp.int32),
            pltpu.VMEM((c0,), jnp.int32),
            pltpu.VMEM((c1,), jnp.int32),
            pltpu.VMEM((c0, 2304), _F32),
            pltpu.VMEM((c1, 640), _F32),
            pltpu.SemaphoreType.DMA,
        ],
    )
    def k(cat0_hbm, cat1_hbm, fm0_hbm, fm1_hbm, out0_hbm, out1_hbm,
          fm0_v, fm1_v, i0_v, i1_v, r0_v, r1_v, sem):
        wid = lax.axis_index("s") * _NC + lax.axis_index("c")
        pltpu.sync_copy(fm0_hbm, fm0_v)
        pltpu.sync_copy(fm1_hbm, fm1_v)
        for c in range(pw0 // c0):
            r0 = wid * pw0 + c * c0
            n = r0 // nf0
            j0 = r0 % nf0
            i0_v[...] = fm0_v[pl.ds(j0, c0)] + n * 384
            pltpu.async_copy(cat0_hbm.at[i0_v], r0_v, sem).wait()
            pltpu.sync_copy(r0_v, out0_hbm.at[pl.ds(r0, c0)])
        for c in range(pw1 // c1):
            r1 = wid * pw1 + c * c1
            n = r1 // nf1
            j0 = r1 % nf1
            for i in range(c1 // 16):
                i1_v[pl.ds(i * 16, 16)] = (
                    fm1_v[pl.ds(j0 + i * 16, 16)] + n * 768)
            pltpu.async_copy(cat1_hbm.at[i1_v], r1_v, sem).wait()
            pltpu.sync_copy(r1_v, out1_hbm.at[pl.ds(r1, c1)])

    return k(cat0, cat1, fm0, fm1)


def _trim_body(in_ref, out_ref):
    out_ref[0] = in_ref[0, :, 0:576]


def _trim_call(a):
    bsz, rows, _ = a.shape
    return pl.pallas_call(
        _trim_body,
        grid=(bsz,),
        in_specs=[pl.BlockSpec((1, rows, 640), lambda b: (b, 0, 0))],
        out_specs=pl.BlockSpec((1, rows, 576), lambda b: (b, 0, 0)),
        out_shape=jax.ShapeDtypeStruct((bsz, rows, 576), _F32),
    )(a)


def kernel(x, W0, W1, W2, W3, W4, fm0, fm1):
    bsz = x.shape[0]

    sel = jnp.zeros((48, 3, 4, 4), _F32)
    for pr in range(4):
        for pc in range(4):
            for c in range(3):
                sel = sel.at[(pr * 4 + pc) * 3 + c, c, pr, pc].set(1.0)
    ph0 = lax.conv_general_dilated(
        x, sel, (4, 4), ((3, 5), (3, 29)),
        dimension_numbers=("NCHW", "OIHW", "NCHW"))
    ph0 = ph0.reshape(bsz, 16, 3, 10192).astype(_BF)

    a0 = jnp.pad(W0.transpose(0, 2, 3, 1).reshape(96, 147),
                 ((0, 32), (0, 0))).astype(_BF)
    w1 = jnp.pad(W1.transpose(2, 3, 1, 0).reshape(9, 96, 192),
                 ((0, 0), (0, 32), (0, 64))).astype(_BF)
    w2 = jnp.pad(W2.transpose(2, 3, 1, 0).reshape(9, 192, 192),
                 ((0, 0), (0, 64), (0, 64))).astype(_BF)
    w3 = jnp.pad(W3.transpose(2, 3, 1, 0).reshape(9, 192, 384),
                 ((0, 0), (0, 64), (0, 0))).astype(_BF)
    w4 = W4.transpose(2, 3, 1, 0).reshape(9, 384, 384).astype(_BF)

    cat0, cat1 = _mega_call(ph0, a0, w1, w2, w3, w4)
    cat0 = cat0.reshape(bsz * 384, 2304)
    cat1 = cat1.reshape(bsz * 768, 640)

    out0, out1p = _sc_gather_both(cat0, cat1, fm0, fm1, bsz)
    out1 = _trim_call(out1p.reshape(bsz, fm1.shape[0], 640))
    return (out0.reshape(bsz, fm0.shape[0], 48, 48),
            out1.reshape(bsz, fm1.shape[0], 24, 24))
